# baseline (speedup 1.0000x reference)
import numpy as np

import jax
import jax.numpy as jnp
from jax.experimental import pallas as pl
from jax.experimental.pallas import tpu as pltpu

N = 1024
R = 4
IN_DIM = 256
HID = 256
D = IN_DIM * R
BF = jnp.bfloat16
F32 = jnp.float32

_perm = (np.arange(D) % R) * IN_DIM + (np.arange(D) // R)
_S_np = np.zeros((D, D), dtype=np.float32)
_S_np[np.arange(D), _perm] = 1.0
_S3_np = np.ascontiguousarray(
    _S_np.reshape(D, R, IN_DIM).transpose(1, 0, 2))


def _body(adj_ref, xf_ref, hf_ref, s_ref, s3_ref,
          wxz_ref, wxr_ref, wxh_ref, whz_ref, whr_ref,
          out_ref, xd_s, hd_s, hd32_s):
    t = pl.program_id(0)

    @pl.when(t == 0)
    def _deint():
        s = s_ref[...]
        xall = jnp.dot(xf_ref[...].astype(BF), s, preferred_element_type=F32)
        hall = jnp.dot(hf_ref[...].astype(BF), s, preferred_element_type=F32)
        for q in range(R):
            cols = slice(q * HID, (q + 1) * HID)
            xd_s[q] = xall[:, cols].astype(BF)
            hd_s[q] = hall[:, cols].astype(BF)
            hd32_s[q] = hall[:, cols]
        out_ref[...] = jnp.zeros((N, HID * R), F32)

    @pl.when(t >= 1)
    def _relation():
        r = jnp.maximum(t - 1, 0)
        a16 = adj_ref[0].astype(BF)
        h32 = hd32_s[r]
        ax = jnp.dot(a16, xd_s[r], preferred_element_type=F32).astype(BF)
        ah = jnp.dot(a16, hd_s[r], preferred_element_type=F32).astype(BF)
        zpre = (jnp.dot(ax, wxz_ref[0].astype(BF), preferred_element_type=F32)
                + jnp.dot(ah, whz_ref[0].astype(BF), preferred_element_type=F32))
        rpre = (jnp.dot(ax, wxr_ref[0].astype(BF), preferred_element_type=F32)
                + jnp.dot(ah, whr_ref[0].astype(BF), preferred_element_type=F32))
        z = jax.nn.sigmoid(zpre)
        rg = jax.nn.sigmoid(rpre)
        tterm = jnp.dot(ax, wxh_ref[0].astype(BF), preferred_element_type=F32)
        g16 = (rg * h32).astype(BF)
        ag = jnp.dot(a16, g16, preferred_element_type=F32)
        ht = jnp.tanh(tterm + jnp.dot(ag.astype(BF), whr_ref[0].astype(BF),
                                      preferred_element_type=F32))
        hn = (z * h32 + (1.0 - z) * ht).astype(BF)
        out_ref[...] += jax.lax.dot_general(
            hn, s3_ref[0], (((1,), (1,)), ((), ())),
            preferred_element_type=F32)


def kernel(X, adj, h_pre, W_xz, W_xr, W_xh, W_hz, W_hr, W_hh):
    del W_hh
    Xf = X.reshape(N, D)
    Hf = h_pre.reshape(N, D)
    S = jnp.asarray(_S_np, dtype=BF)
    S3 = jnp.asarray(_S3_np, dtype=BF)

    def rmap(t):
        return (jnp.maximum(t - 1, 0), 0, 0)

    out = pl.pallas_call(
        _body,
        grid=(R + 1,),
        in_specs=[
            pl.BlockSpec((1, N, N), rmap),
            pl.BlockSpec((N, D), lambda t: (0, 0)),
            pl.BlockSpec((N, D), lambda t: (0, 0)),
            pl.BlockSpec((D, D), lambda t: (0, 0)),
            pl.BlockSpec((1, D, IN_DIM), rmap),
            pl.BlockSpec((1, IN_DIM, HID), rmap),
            pl.BlockSpec((1, IN_DIM, HID), rmap),
            pl.BlockSpec((1, IN_DIM, HID), rmap),
            pl.BlockSpec((1, HID, HID), rmap),
            pl.BlockSpec((1, HID, HID), rmap),
        ],
        out_specs=pl.BlockSpec((N, HID * R), lambda t: (0, 0)),
        out_shape=jax.ShapeDtypeStruct((N, HID * R), F32),
        scratch_shapes=[
            pltpu.VMEM((R, N, IN_DIM), BF),
            pltpu.VMEM((R, N, HID), BF),
            pltpu.VMEM((R, N, HID), F32),
        ],
        compiler_params=pltpu.CompilerParams(
            dimension_semantics=("arbitrary",),
        ),
    )(adj, Xf, Hf, S, S3, W_xz, W_xr, W_xh, W_hz, W_hr)

    return out.reshape(N, HID, R)


# speedup vs baseline: 1.0114x; 1.0114x over previous
import numpy as np

import jax
import jax.numpy as jnp
from jax.experimental import pallas as pl
from jax.experimental.pallas import tpu as pltpu

N = 1024
R = 4
IN_DIM = 256
HID = 256
D = IN_DIM * R
BF = jnp.bfloat16
F32 = jnp.float32

_a = np.arange(D)
_S3_np = np.zeros((R, D, IN_DIM), dtype=np.float32)
for _r in range(R):
    _S3_np[_r, _a[_a % R == _r], (_a[_a % R == _r] // R)] = 1.0


def _body(adj_ref, xf_ref, hf_ref, s3_ref,
          wxz_ref, wxr_ref, wxh_ref, whz_ref, whr_ref, out_ref):
    t = pl.program_id(0)
    s3 = s3_ref[0]
    xd = jnp.dot(xf_ref[...].astype(BF), s3,
                 preferred_element_type=F32).astype(BF)
    h32 = jnp.dot(hf_ref[...].astype(BF), s3, preferred_element_type=F32)
    hd = h32.astype(BF)
    a16 = adj_ref[0].astype(BF)
    ax = jnp.dot(a16, xd, preferred_element_type=F32).astype(BF)
    ah = jnp.dot(a16, hd, preferred_element_type=F32).astype(BF)
    zpre = (jnp.dot(ax, wxz_ref[0].astype(BF), preferred_element_type=F32)
            + jnp.dot(ah, whz_ref[0].astype(BF), preferred_element_type=F32))
    rpre = (jnp.dot(ax, wxr_ref[0].astype(BF), preferred_element_type=F32)
            + jnp.dot(ah, whr_ref[0].astype(BF), preferred_element_type=F32))
    z = jax.nn.sigmoid(zpre)
    rg = jax.nn.sigmoid(rpre)
    tterm = jnp.dot(ax, wxh_ref[0].astype(BF), preferred_element_type=F32)
    g16 = (rg * h32).astype(BF)
    ag = jnp.dot(a16, g16, preferred_element_type=F32)
    ht = jnp.tanh(tterm + jnp.dot(ag.astype(BF), whr_ref[0].astype(BF),
                                  preferred_element_type=F32))
    hn = (z * h32 + (1.0 - z) * ht).astype(BF)
    acc = jax.lax.dot_general(hn, s3, (((1,), (1,)), ((), ())),
                              preferred_element_type=F32)
    out_ref[...] = jnp.where(t == 0, acc, out_ref[...] + acc)


def kernel(X, adj, h_pre, W_xz, W_xr, W_xh, W_hz, W_hr, W_hh):
    del W_hh
    Xf = X.reshape(N, D)
    Hf = h_pre.reshape(N, D)
    S3 = jnp.asarray(_S3_np, dtype=BF)

    def rmap(t):
        return (t, 0, 0)

    out = pl.pallas_call(
        _body,
        grid=(R,),
        in_specs=[
            pl.BlockSpec((1, N, N), rmap),
            pl.BlockSpec((N, D), lambda t: (0, 0)),
            pl.BlockSpec((N, D), lambda t: (0, 0)),
            pl.BlockSpec((1, D, IN_DIM), rmap),
            pl.BlockSpec((1, IN_DIM, HID), rmap),
            pl.BlockSpec((1, IN_DIM, HID), rmap),
            pl.BlockSpec((1, IN_DIM, HID), rmap),
            pl.BlockSpec((1, HID, HID), rmap),
            pl.BlockSpec((1, HID, HID), rmap),
        ],
        out_specs=pl.BlockSpec((N, HID * R), lambda t: (0, 0)),
        out_shape=jax.ShapeDtypeStruct((N, HID * R), F32),
        compiler_params=pltpu.CompilerParams(
            dimension_semantics=("arbitrary",),
        ),
    )(adj, Xf, Hf, S3, W_xz, W_xr, W_xh, W_hz, W_hr)

    return out.reshape(N, HID, R)
